# Initial kernel scaffold; baseline (speedup 1.0000x reference)
import jax
import jax.numpy as jnp
from jax.experimental import pallas as pl


def kernel(data, ent_emb, rel_emb, conv_w, conv_b, bn1_g, bn1_b, bn2_g, bn2_b, fc_w):
    raise NotImplementedError("write your pallas kernel here")


# speedup vs baseline: 1.2764x; 1.2764x over previous
import jax
import jax.numpy as jnp
from jax import lax
from jax.experimental import pallas as pl

_N_ENT = 14541
_HID = 100
_OUT_CH = 32
_E = 10000
_ET = 512
_NT = 20
_EP = _ET * _NT
_DP = 112
_M = _E + _N_ENT
_MC = 2048
_NMC = 12
_MP = _MC * _NMC
_RB = 128
_NRB = 114
_RP = _RB * _NRB
_SENT = 1 << 20


def _score_kernel(he_ref, re_ref, te_ref, params_ref, fc0_ref, fc1_ref, out_ref):
    f32 = jnp.float32
    emask = (lax.broadcasted_iota(jnp.int32, (_EP, 1), 0) < _E).astype(f32)
    h = he_ref[...]
    r = re_ref[...]
    t = te_ref[...]
    cnt1 = 3.0 * _E * _HID
    s1 = jnp.sum(h * emask) + jnp.sum(r * emask) + jnp.sum(t * emask)
    q1 = jnp.sum(h * h * emask) + jnp.sum(r * r * emask) + jnp.sum(t * t * emask)
    mu1 = s1 / cnt1
    var1 = q1 / cnt1 - mu1 * mu1
    g1 = params_ref[0, 0]
    b1 = params_ref[0, 1]
    a1 = g1 * lax.rsqrt(var1 + 1e-5)
    c1 = b1 - mu1 * a1

    def tile_slices(cT):
        hs = he_ref[pl.ds(cT * _ET, _ET), :] * a1 + c1
        rs = re_ref[pl.ds(cT * _ET, _ET), :] * a1 + c1
        ts = te_ref[pl.ds(cT * _ET, _ET), :] * a1 + c1
        return ((hs[:, 0:99], rs[:, 0:99], hs[:, 1:100], rs[:, 1:100]),
                (rs[:, 0:99], ts[:, 0:99], rs[:, 1:100], ts[:, 1:100]))

    w = [params_ref[1, k * _OUT_CH + c] for k in range(4)
         for c in range(_OUT_CH)]

    def stats_body(cT, carry):
        sl0, sl1 = tile_slices(cT)
        m2 = (lax.broadcasted_iota(jnp.int32, (_ET, 1), 0) + cT * _ET < _E
              ).astype(f32)
        out = []
        for c in range(_OUT_CH):
            acc_s = carry[2 * c]
            acc_q = carry[2 * c + 1]
            for sl in (sl0, sl1):
                y = (sl[0] * w[c] + sl[1] * w[32 + c]
                     + sl[2] * w[64 + c] + sl[3] * w[96 + c]) * m2
                acc_s = acc_s + jnp.sum(y)
                acc_q = acc_q + jnp.sum(y * y)
            out.append(acc_s)
            out.append(acc_q)
        return tuple(out)

    zeros = tuple(jnp.zeros((), f32) for _ in range(2 * _OUT_CH))
    stats = lax.fori_loop(0, _NT, stats_body, zeros)
    n2 = 2.0 * _E * 99.0
    a2 = []
    c2 = []
    for c in range(_OUT_CH):
        mean_nb = stats[2 * c] / n2
        var2 = stats[2 * c + 1] / n2 - mean_nb * mean_nb
        ac = params_ref[0, 34 + c] * lax.rsqrt(var2 + 1e-5)
        a2.append(ac)
        c2.append(params_ref[0, 66 + c] + (params_ref[0, 2 + c] - mean_nb) * ac)

    def score_body(cT, carry):
        sl0, sl1 = tile_slices(cT)
        sc = jnp.zeros((_ET, 1), f32)
        for c in range(_OUT_CH):
            for j, sl in enumerate((sl0, sl1)):
                z = jnp.maximum(
                    sl[0] * (w[c] * a2[c]) + sl[1] * (w[32 + c] * a2[c])
                    + sl[2] * (w[64 + c] * a2[c]) + sl[3] * (w[96 + c] * a2[c])
                    + c2[c], 0.0)
                fcrow = (fc0_ref[c:c + 1, :] if j == 0 else fc1_ref[c:c + 1, :])
                sc = sc + jnp.sum(z * fcrow, axis=1, keepdims=True)
        out_ref[pl.ds(cT * _ET, _ET), :] = sc
        return carry

    lax.fori_loop(0, _NT, score_body, 0)


def _agg_kernel(vals_ref, rows_ref, colrows_ref, out_ref):
    f32 = jnp.float32
    b = pl.program_id(0)
    blockrows = b * _RB + lax.broadcasted_iota(jnp.int32, (_RB, 1), 0)

    def max_body(c, m):
        rv = rows_ref[pl.ds(c, 1), :]
        vv = vals_ref[pl.ds(c, 1), :]
        oh = rv == blockrows
        return jnp.maximum(m, jnp.max(jnp.where(oh, vv, -1e30), axis=1,
                                      keepdims=True))

    m = lax.fori_loop(0, _NMC, max_body,
                      jnp.full((_RB, 1), -1e30, f32))

    def sum_body(c, carry):
        zacc, acc = carry
        rv = rows_ref[pl.ds(c, 1), :]
        vv = vals_ref[pl.ds(c, 1), :]
        oh = rv == blockrows
        e = jnp.where(oh, jnp.exp(vv - m), 0.0)
        zacc = zacc + jnp.sum(e, axis=1, keepdims=True)
        cr = colrows_ref[pl.ds(c * _MC, _MC), :]
        acc = acc + jnp.dot(e, cr, preferred_element_type=f32)
        return zacc, acc

    z0 = jnp.zeros((_RB, 1), f32)
    a0 = jnp.zeros((_RB, _DP), f32)
    z, acc = lax.fori_loop(0, _NMC, sum_body, (z0, a0))
    out_ref[...] = acc * (1.0 / jnp.where(z > 0.0, z, 1.0))


def kernel(data, ent_emb, rel_emb, conv_w, conv_b, bn1_g, bn1_b, bn2_g, bn2_b, fc_w):
    h = data[:, 0].astype(jnp.int32)
    r = data[:, 1].astype(jnp.int32)
    t = data[:, 2].astype(jnp.int32)
    ent_p = jnp.pad(ent_emb, ((0, 0), (0, _DP - _HID)))
    rel_p = jnp.pad(rel_emb, ((0, 0), (0, _DP - _HID)))

    hp = jnp.pad(h, (0, _EP - _E))
    rp = jnp.pad(r, (0, _EP - _E))
    tp = jnp.pad(t, (0, _EP - _E))
    he = ent_p[hp]
    re_ = rel_p[rp]
    te = ent_p[tp]

    row0 = jnp.concatenate([bn1_g, bn1_b, conv_b, bn2_g, bn2_b,
                            jnp.zeros((30,), jnp.float32)])
    w4 = conv_w.reshape(_OUT_CH, 4)
    row1 = jnp.concatenate([w4[:, 0], w4[:, 1], w4[:, 2], w4[:, 3]])
    params = jnp.stack([row0, row1])
    fcr = fc_w.reshape(_OUT_CH, 99, 2)
    fc0 = fcr[:, :, 0] + 0.0
    fc1 = fcr[:, :, 1] + 0.0

    scores2d = pl.pallas_call(
        _score_kernel,
        out_shape=jax.ShapeDtypeStruct((_EP, 1), jnp.float32),
    )(he, re_, te, params, fc0, fc1)
    score = scores2d.reshape(-1)[:_E]

    rows = jnp.concatenate([h, jnp.arange(_N_ENT, dtype=jnp.int32)])
    cols = jnp.concatenate([t, jnp.arange(_N_ENT, dtype=jnp.int32)])
    vals = jnp.concatenate([score, jnp.ones((_N_ENT,), jnp.float32)])
    keys = rows * _N_ENT + cols
    order = jnp.argsort(keys)
    keys_s = keys[order]
    vals_s = vals[order]
    new_seg = jnp.concatenate([jnp.ones((1,), jnp.bool_),
                               keys_s[1:] != keys_s[:-1]])
    seg = jnp.cumsum(new_seg.astype(jnp.int32)) - 1
    coal = jax.ops.segment_sum(vals_s, seg, num_segments=_M)
    uniq_key = jax.ops.segment_min(keys_s, seg, num_segments=_M)
    n_uniq = seg[-1] + 1
    valid = jnp.arange(_M, dtype=jnp.int32) < n_uniq
    row_ids = jnp.where(valid, uniq_key // _N_ENT, _SENT).astype(jnp.int32)
    col_ids = jnp.where(valid, uniq_key % _N_ENT, 0).astype(jnp.int32)
    coal = jnp.where(valid, coal, 0.0)

    rows2d = jnp.pad(row_ids, (0, _MP - _M),
                     constant_values=_SENT).reshape(_NMC, _MC)
    vals2d = jnp.pad(coal, (0, _MP - _M)).reshape(_NMC, _MC)
    colp = jnp.pad(col_ids, (0, _MP - _M))
    colrows = ent_p[colp]

    out = pl.pallas_call(
        _agg_kernel,
        grid=(_NRB,),
        in_specs=[
            pl.BlockSpec((_NMC, _MC), lambda b: (0, 0)),
            pl.BlockSpec((_NMC, _MC), lambda b: (0, 0)),
            pl.BlockSpec((_MP, _DP), lambda b: (0, 0)),
        ],
        out_specs=pl.BlockSpec((_RB, _DP), lambda b: (b, 0)),
        out_shape=jax.ShapeDtypeStruct((_RP, _DP), jnp.float32),
    )(vals2d, rows2d, colrows)
    return out[:_N_ENT, :_HID]


# speedup vs baseline: 1.3411x; 1.0507x over previous
import jax
import jax.numpy as jnp
from jax import lax
from jax.experimental import pallas as pl

_N_ENT = 14541
_HID = 100
_OUT_CH = 32
_E = 10000
_ET = 512
_NT = 20
_EP = _ET * _NT
_DP = 112
_M = _E + _N_ENT
_MC = 512
_NMC = 48
_MP = _MC * _NMC
_RB = 128
_NRB = 114
_RP = _RB * _NRB
_SENT = 1 << 20


def _score_kernel(he_ref, re_ref, te_ref, params_ref, fc0_ref, fc1_ref, out_ref):
    f32 = jnp.float32
    emask = (lax.broadcasted_iota(jnp.int32, (_EP, 1), 0) < _E).astype(f32)
    h = he_ref[...]
    r = re_ref[...]
    t = te_ref[...]
    cnt1 = 3.0 * _E * _HID
    s1 = jnp.sum(h * emask) + jnp.sum(r * emask) + jnp.sum(t * emask)
    q1 = jnp.sum(h * h * emask) + jnp.sum(r * r * emask) + jnp.sum(t * t * emask)
    mu1 = s1 / cnt1
    var1 = q1 / cnt1 - mu1 * mu1
    g1 = params_ref[0, 0]
    b1 = params_ref[0, 1]
    a1 = g1 * lax.rsqrt(var1 + 1e-5)
    c1 = b1 - mu1 * a1

    def tile_slices(cT):
        hs = he_ref[pl.ds(cT * _ET, _ET), :] * a1 + c1
        rs = re_ref[pl.ds(cT * _ET, _ET), :] * a1 + c1
        ts = te_ref[pl.ds(cT * _ET, _ET), :] * a1 + c1
        return ((hs[:, 0:99], rs[:, 0:99], hs[:, 1:100], rs[:, 1:100]),
                (rs[:, 0:99], ts[:, 0:99], rs[:, 1:100], ts[:, 1:100]))

    w = [params_ref[1, k * _OUT_CH + c] for k in range(4)
         for c in range(_OUT_CH)]

    def stats_body(cT, carry):
        sl0, sl1 = tile_slices(cT)
        m2 = (lax.broadcasted_iota(jnp.int32, (_ET, 1), 0) + cT * _ET < _E
              ).astype(f32)
        out = []
        for c in range(_OUT_CH):
            acc_s = carry[2 * c]
            acc_q = carry[2 * c + 1]
            for sl in (sl0, sl1):
                y = (sl[0] * w[c] + sl[1] * w[32 + c]
                     + sl[2] * w[64 + c] + sl[3] * w[96 + c]) * m2
                acc_s = acc_s + jnp.sum(y)
                acc_q = acc_q + jnp.sum(y * y)
            out.append(acc_s)
            out.append(acc_q)
        return tuple(out)

    zeros = tuple(jnp.zeros((), f32) for _ in range(2 * _OUT_CH))
    stats = lax.fori_loop(0, _NT, stats_body, zeros)
    n2 = 2.0 * _E * 99.0
    a2 = []
    c2 = []
    for c in range(_OUT_CH):
        mean_nb = stats[2 * c] / n2
        var2 = stats[2 * c + 1] / n2 - mean_nb * mean_nb
        ac = params_ref[0, 34 + c] * lax.rsqrt(var2 + 1e-5)
        a2.append(ac)
        c2.append(params_ref[0, 66 + c] + (params_ref[0, 2 + c] - mean_nb) * ac)

    def score_body(cT, carry):
        sl0, sl1 = tile_slices(cT)
        sc = jnp.zeros((_ET, 1), f32)
        for c in range(_OUT_CH):
            for j, sl in enumerate((sl0, sl1)):
                z = jnp.maximum(
                    sl[0] * (w[c] * a2[c]) + sl[1] * (w[32 + c] * a2[c])
                    + sl[2] * (w[64 + c] * a2[c]) + sl[3] * (w[96 + c] * a2[c])
                    + c2[c], 0.0)
                fcrow = (fc0_ref[c:c + 1, :] if j == 0 else fc1_ref[c:c + 1, :])
                sc = sc + jnp.sum(z * fcrow, axis=1, keepdims=True)
        out_ref[pl.ds(cT * _ET, _ET), :] = sc
        return carry

    lax.fori_loop(0, _NT, score_body, 0)


def _agg_kernel(vals_ref, rows_ref, colrows_ref, cfirst_ref, clast_ref, out_ref):
    f32 = jnp.float32
    b = pl.program_id(0)
    bmin = b * _RB
    blockrows = bmin + lax.broadcasted_iota(jnp.int32, (_RB, 1), 0)

    def overlaps(c):
        return jnp.logical_and(cfirst_ref[c, 0] <= bmin + (_RB - 1),
                               clast_ref[c, 0] >= bmin)

    def max_body(c, m):
        def live(m):
            rv = rows_ref[pl.ds(c, 1), :]
            vv = vals_ref[pl.ds(c, 1), :]
            oh = rv == blockrows
            return jnp.maximum(m, jnp.max(jnp.where(oh, vv, -1e30), axis=1,
                                          keepdims=True))
        return lax.cond(overlaps(c), live, lambda m: m, m)

    m = lax.fori_loop(0, _NMC, max_body,
                      jnp.full((_RB, 1), -1e30, f32))

    def sum_body(c, carry):
        def live(carry):
            zacc, acc = carry
            rv = rows_ref[pl.ds(c, 1), :]
            vv = vals_ref[pl.ds(c, 1), :]
            oh = rv == blockrows
            e = jnp.where(oh, jnp.exp(vv - m), 0.0)
            zacc = zacc + jnp.sum(e, axis=1, keepdims=True)
            cr = colrows_ref[pl.ds(c * _MC, _MC), :]
            acc = acc + jnp.dot(e, cr, preferred_element_type=f32)
            return zacc, acc
        return lax.cond(overlaps(c), live, lambda x: x, carry)

    z0 = jnp.zeros((_RB, 1), f32)
    a0 = jnp.zeros((_RB, _DP), f32)
    z, acc = lax.fori_loop(0, _NMC, sum_body, (z0, a0))
    out_ref[...] = acc * (1.0 / jnp.where(z > 0.0, z, 1.0))


def kernel(data, ent_emb, rel_emb, conv_w, conv_b, bn1_g, bn1_b, bn2_g, bn2_b, fc_w):
    h = data[:, 0].astype(jnp.int32)
    r = data[:, 1].astype(jnp.int32)
    t = data[:, 2].astype(jnp.int32)
    ent_p = jnp.pad(ent_emb, ((0, 0), (0, _DP - _HID)))
    rel_p = jnp.pad(rel_emb, ((0, 0), (0, _DP - _HID)))

    hp = jnp.pad(h, (0, _EP - _E))
    rp = jnp.pad(r, (0, _EP - _E))
    tp = jnp.pad(t, (0, _EP - _E))
    he = ent_p[hp]
    re_ = rel_p[rp]
    te = ent_p[tp]

    row0 = jnp.concatenate([bn1_g, bn1_b, conv_b, bn2_g, bn2_b,
                            jnp.zeros((30,), jnp.float32)])
    w4 = conv_w.reshape(_OUT_CH, 4)
    row1 = jnp.concatenate([w4[:, 0], w4[:, 1], w4[:, 2], w4[:, 3]])
    params = jnp.stack([row0, row1])
    fcr = fc_w.reshape(_OUT_CH, 99, 2)
    fc0 = fcr[:, :, 0] + 0.0
    fc1 = fcr[:, :, 1] + 0.0

    scores2d = pl.pallas_call(
        _score_kernel,
        out_shape=jax.ShapeDtypeStruct((_EP, 1), jnp.float32),
    )(he, re_, te, params, fc0, fc1)
    score = scores2d.reshape(-1)[:_E]

    rows = jnp.concatenate([h, jnp.arange(_N_ENT, dtype=jnp.int32)])
    cols = jnp.concatenate([t, jnp.arange(_N_ENT, dtype=jnp.int32)])
    vals = jnp.concatenate([score, jnp.ones((_N_ENT,), jnp.float32)])
    keys = rows * _N_ENT + cols
    order = jnp.argsort(keys)
    keys_s = keys[order]
    vals_s = vals[order]
    new_seg = jnp.concatenate([jnp.ones((1,), jnp.bool_),
                               keys_s[1:] != keys_s[:-1]])
    seg = jnp.cumsum(new_seg.astype(jnp.int32)) - 1
    coal = jax.ops.segment_sum(vals_s, seg, num_segments=_M)
    uniq_key = jax.ops.segment_min(keys_s, seg, num_segments=_M)
    n_uniq = seg[-1] + 1
    valid = jnp.arange(_M, dtype=jnp.int32) < n_uniq
    row_ids = jnp.where(valid, uniq_key // _N_ENT, _SENT).astype(jnp.int32)
    col_ids = jnp.where(valid, uniq_key % _N_ENT, 0).astype(jnp.int32)
    coal = jnp.where(valid, coal, 0.0)

    rows2d = jnp.pad(row_ids, (0, _MP - _M),
                     constant_values=_SENT).reshape(_NMC, _MC)
    vals2d = jnp.pad(coal, (0, _MP - _M)).reshape(_NMC, _MC)
    colp = jnp.pad(col_ids, (0, _MP - _M))
    colrows = ent_p[colp]
    cfirst = jnp.broadcast_to(rows2d[:, 0:1], (_NMC, 128)) + 0
    clast = jnp.broadcast_to(rows2d[:, _MC - 1:_MC], (_NMC, 128)) + 0

    out = pl.pallas_call(
        _agg_kernel,
        grid=(_NRB,),
        in_specs=[
            pl.BlockSpec((_NMC, _MC), lambda b: (0, 0)),
            pl.BlockSpec((_NMC, _MC), lambda b: (0, 0)),
            pl.BlockSpec((_MP, _DP), lambda b: (0, 0)),
            pl.BlockSpec((_NMC, 128), lambda b: (0, 0)),
            pl.BlockSpec((_NMC, 128), lambda b: (0, 0)),
        ],
        out_specs=pl.BlockSpec((_RB, _DP), lambda b: (b, 0)),
        out_shape=jax.ShapeDtypeStruct((_RP, _DP), jnp.float32),
    )(vals2d, rows2d, colrows, cfirst, clast)
    return out[:_N_ENT, :_HID]


# speedup vs baseline: 1.4102x; 1.0515x over previous
import jax
import jax.numpy as jnp
from jax import lax
from jax.experimental import pallas as pl

_N_ENT = 14541
_HID = 100
_OUT_CH = 32
_E = 10000
_ET = 512
_NT = 20
_EP = _ET * _NT
_DP = 112
_M = _E + _N_ENT
_MC = 512
_NMC = 48
_MP = _MC * _NMC
_RB = 128
_NRB = 114
_RP = _RB * _NRB
_SENT = 1 << 20


def _score_kernel(he_ref, re_ref, te_ref, params_ref, fc0_ref, fc1_ref, out_ref):
    f32 = jnp.float32
    emask = (lax.broadcasted_iota(jnp.int32, (_EP, 1), 0) < _E).astype(f32)
    h = he_ref[...]
    r = re_ref[...]
    t = te_ref[...]
    cnt1 = 3.0 * _E * _HID
    s1 = jnp.sum(h * emask) + jnp.sum(r * emask) + jnp.sum(t * emask)
    q1 = jnp.sum(h * h * emask) + jnp.sum(r * r * emask) + jnp.sum(t * t * emask)
    mu1 = s1 / cnt1
    var1 = q1 / cnt1 - mu1 * mu1
    g1 = params_ref[0, 0]
    b1 = params_ref[0, 1]
    a1 = g1 * lax.rsqrt(var1 + 1e-5)
    c1 = b1 - mu1 * a1

    def tile_slices(cT):
        hs = he_ref[pl.ds(cT * _ET, _ET), :] * a1 + c1
        rs = re_ref[pl.ds(cT * _ET, _ET), :] * a1 + c1
        ts = te_ref[pl.ds(cT * _ET, _ET), :] * a1 + c1
        return ((hs[:, 0:99], rs[:, 0:99], hs[:, 1:100], rs[:, 1:100]),
                (rs[:, 0:99], ts[:, 0:99], rs[:, 1:100], ts[:, 1:100]))

    w = [params_ref[1, k * _OUT_CH + c] for k in range(4)
         for c in range(_OUT_CH)]

    def stats_body(cT, carry):
        sl0, sl1 = tile_slices(cT)
        m2 = (lax.broadcasted_iota(jnp.int32, (_ET, 1), 0) + cT * _ET < _E
              ).astype(f32)
        out = []
        for c in range(_OUT_CH):
            acc_s = carry[2 * c]
            acc_q = carry[2 * c + 1]
            for sl in (sl0, sl1):
                y = (sl[0] * w[c] + sl[1] * w[32 + c]
                     + sl[2] * w[64 + c] + sl[3] * w[96 + c]) * m2
                acc_s = acc_s + jnp.sum(y)
                acc_q = acc_q + jnp.sum(y * y)
            out.append(acc_s)
            out.append(acc_q)
        return tuple(out)

    zeros = tuple(jnp.zeros((), f32) for _ in range(2 * _OUT_CH))
    stats = lax.fori_loop(0, _NT, stats_body, zeros)
    n2 = 2.0 * _E * 99.0
    a2 = []
    c2 = []
    for c in range(_OUT_CH):
        mean_nb = stats[2 * c] / n2
        var2 = stats[2 * c + 1] / n2 - mean_nb * mean_nb
        ac = params_ref[0, 34 + c] * lax.rsqrt(var2 + 1e-5)
        a2.append(ac)
        c2.append(params_ref[0, 66 + c] + (params_ref[0, 2 + c] - mean_nb) * ac)

    def score_body(cT, carry):
        sl0, sl1 = tile_slices(cT)
        acc = jnp.zeros((_ET, 99), f32)
        for c in range(_OUT_CH):
            for j, sl in enumerate((sl0, sl1)):
                z = jnp.maximum(
                    sl[0] * (w[c] * a2[c]) + sl[1] * (w[32 + c] * a2[c])
                    + sl[2] * (w[64 + c] * a2[c]) + sl[3] * (w[96 + c] * a2[c])
                    + c2[c], 0.0)
                fcrow = (fc0_ref[c:c + 1, :] if j == 0 else fc1_ref[c:c + 1, :])
                acc = acc + z * fcrow
        out_ref[pl.ds(cT * _ET, _ET), :] = jnp.sum(acc, axis=1, keepdims=True)
        return carry

    lax.fori_loop(0, _NT, score_body, 0)


def _agg_kernel(vals_ref, rows_ref, colrows_ref, cfirst_ref, clast_ref, out_ref):
    f32 = jnp.float32
    b = pl.program_id(0)
    bmin = b * _RB
    blockrows = bmin + lax.broadcasted_iota(jnp.int32, (_RB, 1), 0)

    def overlaps(c):
        return jnp.logical_and(cfirst_ref[c, 0] <= bmin + (_RB - 1),
                               clast_ref[c, 0] >= bmin)

    def max_body(c, m):
        def live(m):
            rv = rows_ref[pl.ds(c, 1), :]
            vv = vals_ref[pl.ds(c, 1), :]
            oh = rv == blockrows
            return jnp.maximum(m, jnp.max(jnp.where(oh, vv, -1e30), axis=1,
                                          keepdims=True))
        return lax.cond(overlaps(c), live, lambda m: m, m)

    m = lax.fori_loop(0, _NMC, max_body,
                      jnp.full((_RB, 1), -1e30, f32))

    def sum_body(c, carry):
        def live(carry):
            zacc, acc = carry
            rv = rows_ref[pl.ds(c, 1), :]
            vv = vals_ref[pl.ds(c, 1), :]
            oh = rv == blockrows
            e = jnp.where(oh, jnp.exp(vv - m), 0.0)
            zacc = zacc + jnp.sum(e, axis=1, keepdims=True)
            cr = colrows_ref[pl.ds(c * _MC, _MC), :]
            acc = acc + jnp.dot(e, cr, preferred_element_type=f32)
            return zacc, acc
        return lax.cond(overlaps(c), live, lambda x: x, carry)

    z0 = jnp.zeros((_RB, 1), f32)
    a0 = jnp.zeros((_RB, _DP), f32)
    z, acc = lax.fori_loop(0, _NMC, sum_body, (z0, a0))
    out_ref[...] = acc * (1.0 / jnp.where(z > 0.0, z, 1.0))


def kernel(data, ent_emb, rel_emb, conv_w, conv_b, bn1_g, bn1_b, bn2_g, bn2_b, fc_w):
    h = data[:, 0].astype(jnp.int32)
    r = data[:, 1].astype(jnp.int32)
    t = data[:, 2].astype(jnp.int32)
    ent_p = jnp.pad(ent_emb, ((0, 0), (0, _DP - _HID)))
    rel_p = jnp.pad(rel_emb, ((0, 0), (0, _DP - _HID)))

    hp = jnp.pad(h, (0, _EP - _E))
    rp = jnp.pad(r, (0, _EP - _E))
    tp = jnp.pad(t, (0, _EP - _E))
    he = ent_p[hp]
    re_ = rel_p[rp]
    te = ent_p[tp]

    row0 = jnp.concatenate([bn1_g, bn1_b, conv_b, bn2_g, bn2_b,
                            jnp.zeros((30,), jnp.float32)])
    w4 = conv_w.reshape(_OUT_CH, 4)
    row1 = jnp.concatenate([w4[:, 0], w4[:, 1], w4[:, 2], w4[:, 3]])
    params = jnp.stack([row0, row1])
    fcr = fc_w.reshape(_OUT_CH, 99, 2)
    fc0 = fcr[:, :, 0] + 0.0
    fc1 = fcr[:, :, 1] + 0.0

    scores2d = pl.pallas_call(
        _score_kernel,
        out_shape=jax.ShapeDtypeStruct((_EP, 1), jnp.float32),
    )(he, re_, te, params, fc0, fc1)
    score = scores2d.reshape(-1)[:_E]

    rows = jnp.concatenate([h, jnp.arange(_N_ENT, dtype=jnp.int32)])
    cols = jnp.concatenate([t, jnp.arange(_N_ENT, dtype=jnp.int32)])
    vals = jnp.concatenate([score, jnp.ones((_N_ENT,), jnp.float32)])
    keys = rows * _N_ENT + cols
    order = jnp.argsort(keys)
    keys_s = keys[order]
    vals_s = vals[order]
    new_seg = jnp.concatenate([jnp.ones((1,), jnp.bool_),
                               keys_s[1:] != keys_s[:-1]])
    seg = jnp.cumsum(new_seg.astype(jnp.int32)) - 1
    coal = jax.ops.segment_sum(vals_s, seg, num_segments=_M)
    uniq_key = jax.ops.segment_min(keys_s, seg, num_segments=_M)
    n_uniq = seg[-1] + 1
    valid = jnp.arange(_M, dtype=jnp.int32) < n_uniq
    row_ids = jnp.where(valid, uniq_key // _N_ENT, _SENT).astype(jnp.int32)
    col_ids = jnp.where(valid, uniq_key % _N_ENT, 0).astype(jnp.int32)
    coal = jnp.where(valid, coal, 0.0)

    rows2d = jnp.pad(row_ids, (0, _MP - _M),
                     constant_values=_SENT).reshape(_NMC, _MC)
    vals2d = jnp.pad(coal, (0, _MP - _M)).reshape(_NMC, _MC)
    colp = jnp.pad(col_ids, (0, _MP - _M))
    colrows = ent_p[colp]
    cfirst = jnp.broadcast_to(rows2d[:, 0:1], (_NMC, 128)) + 0
    clast = jnp.broadcast_to(rows2d[:, _MC - 1:_MC], (_NMC, 128)) + 0

    out = pl.pallas_call(
        _agg_kernel,
        grid=(_NRB,),
        in_specs=[
            pl.BlockSpec((_NMC, _MC), lambda b: (0, 0)),
            pl.BlockSpec((_NMC, _MC), lambda b: (0, 0)),
            pl.BlockSpec((_MP, _DP), lambda b: (0, 0)),
            pl.BlockSpec((_NMC, 128), lambda b: (0, 0)),
            pl.BlockSpec((_NMC, 128), lambda b: (0, 0)),
        ],
        out_specs=pl.BlockSpec((_RB, _DP), lambda b: (b, 0)),
        out_shape=jax.ShapeDtypeStruct((_RP, _DP), jnp.float32),
    )(vals2d, rows2d, colrows, cfirst, clast)
    return out[:_N_ENT, :_HID]
